# baseline (speedup 1.0000x reference)
import functools

import jax
import jax.numpy as jnp
import numpy as np
from jax import lax
from jax.experimental import pallas as pl
from jax.experimental.pallas import tpu as pltpu
from jax.experimental.pallas import tpu_sc as plsc

USER_N = 60000
ITEM_N = 40000
NN = USER_N + ITEM_N
EDGES = 3200000
H = 16
SLOPE = 0.2

NC = 2
NS = 16
NW = NC * NS
CHUNK = 128
NCHUNKS = EDGES // CHUNK
BASE_CPW = NCHUNKS // NW
EXTRA = NCHUNKS - NW * BASE_CPW
SUP = 16
FULL_STEPS = BASE_CPW // SUP
TAIL = BASE_CPW - FULL_STEPS * SUP
NBUF = 8
NPAD = 102400
RPS = NPAD // NS
GROWS = NBUF * CHUNK

_BC_DNUMS = lax.GatherDimensionNumbers(
    offset_dims=(), collapsed_slice_dims=(0,), start_index_map=(0,))


def _bcast_lane(vec, lane):
    idx = jnp.full((16, 1), lane, jnp.int32)
    return lax.gather(vec, idx, _BC_DNUMS, (1,),
                      mode=lax.GatherScatterMode.PROMISE_IN_BOUNDS)


def _spmm_sc(rows2d, cols2d, vals2d, feat):
    mesh = plsc.VectorSubcoreMesh(core_axis_name="c", subcore_axis_name="s")

    @functools.partial(
        pl.kernel,
        out_type=jax.ShapeDtypeStruct((NC, NPAD, H), jnp.float32),
        mesh=mesh,
        scratch_types=[
            pltpu.VMEM((SUP, CHUNK), jnp.int32),
            pltpu.VMEM((SUP, CHUNK), jnp.int32),
            pltpu.VMEM((SUP, CHUNK), jnp.float32),
            pltpu.VMEM((GROWS, H), jnp.float32),
            pltpu.VMEM_SHARED((NPAD, H), jnp.float32),
            pltpu.SemaphoreType.DMA((NBUF,)),
            pltpu.SemaphoreType.DMA((NBUF,)),
        ],
        compiler_params=pltpu.CompilerParams(use_tc_tiling_on_sc=False),
    )
    def k(rows_hbm, cols_hbm, vals_hbm, feat_hbm, out_hbm,
          row_v, col_v, val_v, gat_v, acc, gsem, ssem):
        cid = lax.axis_index("c")
        sid = lax.axis_index("s")
        wid = cid * NS + sid

        @pl.loop(0, GROWS)
        def _(g):
            gat_v[g, :] = jnp.zeros((16,), jnp.float32)

        for t in range(RPS // GROWS):
            pltpu.sync_copy(gat_v, acc.at[pl.ds(sid * RPS + t * GROWS, GROWS)])
        rem = RPS % GROWS
        if rem:
            pltpu.sync_copy(
                gat_v.at[pl.ds(0, rem)],
                acc.at[pl.ds(sid * RPS + (RPS // GROWS) * GROWS, rem)])
        plsc.subcore_barrier()

        def process_staged(count):
            for r0 in range(0, count, NBUF):
                n = min(NBUF, count - r0)
                gets = [
                    pltpu.async_copy(feat_hbm.at[col_v.at[r0 + u]],
                                     gat_v.at[pl.ds(u * CHUNK, CHUNK)],
                                     gsem.at[u])
                    for u in range(n)
                ]
                puts = []
                for u in range(n):
                    j = r0 + u
                    gets[u].wait()

                    @pl.loop(0, CHUNK // 16, step=4)
                    def _(g, u=u, j=j):
                        for dg in range(4):
                            vv = val_v[j, pl.ds((g + dg) * 16, 16)]
                            for l in range(16):
                                bc = _bcast_lane(vv, l)
                                kk = u * CHUNK + (g + dg) * 16 + l
                                gat_v[kk, :] = gat_v[kk, :] * bc

                    puts.append(
                        pltpu.async_copy(gat_v.at[pl.ds(u * CHUNK, CHUNK)],
                                         acc.at[row_v.at[j]],
                                         ssem.at[u], add=True))
                for p in puts:
                    p.wait()

        lo = wid * BASE_CPW + jnp.minimum(wid, EXTRA)

        @pl.loop(0, FULL_STEPS)
        def _(si):
            c0 = lo + si * SUP
            pltpu.sync_copy(rows_hbm.at[pl.ds(c0, SUP)], row_v)
            pltpu.sync_copy(cols_hbm.at[pl.ds(c0, SUP)], col_v)
            pltpu.sync_copy(vals_hbm.at[pl.ds(c0, SUP)], val_v)
            process_staged(SUP)

        ct = lo + FULL_STEPS * SUP
        pltpu.sync_copy(rows_hbm.at[pl.ds(ct, TAIL)], row_v.at[pl.ds(0, TAIL)])
        pltpu.sync_copy(cols_hbm.at[pl.ds(ct, TAIL)], col_v.at[pl.ds(0, TAIL)])
        pltpu.sync_copy(vals_hbm.at[pl.ds(ct, TAIL)], val_v.at[pl.ds(0, TAIL)])
        process_staged(TAIL)

        @pl.when(wid < EXTRA)
        def _():
            ce = lo + BASE_CPW
            pltpu.sync_copy(rows_hbm.at[pl.ds(ce, 1)], row_v.at[pl.ds(0, 1)])
            pltpu.sync_copy(cols_hbm.at[pl.ds(ce, 1)], col_v.at[pl.ds(0, 1)])
            pltpu.sync_copy(vals_hbm.at[pl.ds(ce, 1)], val_v.at[pl.ds(0, 1)])
            process_staged(1)

        plsc.subcore_barrier()
        pltpu.sync_copy(acc.at[pl.ds(sid * RPS, RPS)],
                        out_hbm.at[cid, pl.ds(sid * RPS, RPS)])

    return k(rows2d, cols2d, vals2d, feat)


PU = USER_N // 8
PI = ITEM_N // 8
PBLK = 512
PUP = 7680
PIP = 5120
UBLK = PUP // PBLK
NBLK = UBLK + PIP // PBLK
PT = NPAD // 8
ISHIFT = PUP * 8 - USER_N
_HP = lax.Precision.HIGHEST


def _seg16():
    r = lax.broadcasted_iota(jnp.int32, (128, 128), 0) // H
    c = lax.broadcasted_iota(jnp.int32, (128, 128), 1) // H
    return jnp.where(r == c, 1.0, 0.0).astype(jnp.float32)


def _kron8(w):
    eye = jnp.eye(8, dtype=jnp.float32)
    return jnp.einsum("ab,kc->akbc", eye, w).reshape(128, 128)


def _transform_tc(user_p, item_p, wu, wv):
    def body(u_ref, v_ref, wu_ref, wv_ref, out_ref):
        i = pl.program_id(0)
        x = jnp.where(i < UBLK, u_ref[...], v_ref[...])
        w = jnp.where(i < UBLK, wu_ref[...], wv_ref[...])
        out_ref[...] = jnp.dot(x, w, preferred_element_type=jnp.float32,
                               precision=_HP)

    return pl.pallas_call(
        body,
        grid=(NBLK,),
        in_specs=[
            pl.BlockSpec((PBLK, 128), lambda i: (jnp.minimum(i, UBLK - 1), 0)),
            pl.BlockSpec((PBLK, 128), lambda i: (jnp.maximum(i - UBLK, 0), 0)),
            pl.BlockSpec((128, 128), lambda i: (0, 0)),
            pl.BlockSpec((128, 128), lambda i: (0, 0)),
        ],
        out_specs=pl.BlockSpec((PBLK, 128), lambda i: (i, 0)),
        out_shape=jax.ShapeDtypeStruct((PT, 128), jnp.float32),
    )(user_p, item_p, wu, wv)


def _leaky_norm(s, seg):
    emb = jnp.where(s >= 0, s, SLOPE * s)
    n2 = jnp.dot(emb * emb, seg, preferred_element_type=jnp.float32,
                 precision=_HP)
    nrm = emb / jnp.maximum(jnp.sqrt(n2), 1e-12)
    return emb, nrm


def _combine_tc(p_p, wu, wv):
    def body(p_ref, wu_ref, wv_ref, nrm_ref, feat_ref):
        i = pl.program_id(0)
        emb, nrm = _leaky_norm(p_ref[0] + p_ref[1], _seg16())
        nrm_ref[...] = nrm
        w = jnp.where(i < UBLK, wu_ref[...], wv_ref[...])
        feat_ref[...] = jnp.dot(emb, w, preferred_element_type=jnp.float32,
                                precision=_HP)

    return pl.pallas_call(
        body,
        grid=(NBLK,),
        in_specs=[
            pl.BlockSpec((NC, PBLK, 128), lambda i: (0, i, 0)),
            pl.BlockSpec((128, 128), lambda i: (0, 0)),
            pl.BlockSpec((128, 128), lambda i: (0, 0)),
        ],
        out_specs=[pl.BlockSpec((PBLK, 128), lambda i: (i, 0)),
                   pl.BlockSpec((PBLK, 128), lambda i: (i, 0))],
        out_shape=[jax.ShapeDtypeStruct((PT, 128), jnp.float32),
                   jax.ShapeDtypeStruct((PT, 128), jnp.float32)],
    )(p_p, wu, wv)


def _final_tc(p_p, nrm1_p, emb_part_p, smat, blk0, nblk):
    def body(p_ref, n1_ref, e_ref, s_ref, out_ref):
        _, nrm2 = _leaky_norm(p_ref[0] + p_ref[1], _seg16())
        hp = dict(preferred_element_type=jnp.float32, precision=_HP)
        out_ref[...] = (jnp.dot(e_ref[...], s_ref[0], **hp) +
                        jnp.dot(n1_ref[...], s_ref[1], **hp) +
                        jnp.dot(nrm2, s_ref[2], **hp))

    return pl.pallas_call(
        body,
        grid=(nblk,),
        in_specs=[
            pl.BlockSpec((NC, PBLK, 128), lambda i: (0, i + blk0, 0)),
            pl.BlockSpec((PBLK, 128), lambda i: (i + blk0, 0)),
            pl.BlockSpec((PBLK, 128), lambda i: (i, 0)),
            pl.BlockSpec((3, 128, 384), lambda i: (0, 0, 0)),
        ],
        out_specs=pl.BlockSpec((PBLK, 384), lambda i: (i, 0)),
        out_shape=jax.ShapeDtypeStruct((nblk * PBLK, 384), jnp.float32),
    )(p_p, nrm1_p, emb_part_p, smat)


def _smat():
    m = np.zeros((3, 128, 384), np.float32)
    for t in range(3):
        for mm in range(8):
            for c in range(H):
                m[t, mm * H + c, mm * 3 * H + t * H + c] = 1.0
    return jnp.asarray(m)


def kernel(adj_indices, adj_values, user_emb, item_emb, u_w0, v_w0, u_w1, v_w1):
    idx = adj_indices + jnp.where(adj_indices >= USER_N, ISHIFT, 0)
    rows2d = idx[0].reshape(NCHUNKS, CHUNK)
    cols2d = idx[1].reshape(NCHUNKS, CHUNK)
    vals2d = adj_values.reshape(NCHUNKS, CHUNK)

    zu = jnp.zeros((PUP - PU, 128), jnp.float32)
    zi = jnp.zeros((PIP - PI, 128), jnp.float32)
    user_p = jnp.concatenate([user_emb.reshape(PU, 128), zu], axis=0)
    item_p = jnp.concatenate([item_emb.reshape(PI, 128), zi], axis=0)
    w0u, w0v, w1u, w1v = (_kron8(w) for w in (u_w0, v_w0, u_w1, v_w1))
    smat = _smat()

    table0 = _transform_tc(user_p, item_p, w0u, w0v)

    p1 = _spmm_sc(rows2d, cols2d, vals2d, table0.reshape(NPAD, H))
    nrm1, table1 = _combine_tc(p1.reshape(NC, PT, 128), w1u, w1v)

    p2 = _spmm_sc(rows2d, cols2d, vals2d, table1.reshape(NPAD, H))
    p2_p = p2.reshape(NC, PT, 128)

    user_embedding = _final_tc(p2_p, nrm1, user_p, smat, 0, UBLK)
    item_embedding = _final_tc(p2_p, nrm1, item_p, smat, UBLK, NBLK - UBLK)
    return (user_embedding.reshape(PUP * 8, 3 * H)[:USER_N],
            item_embedding.reshape(PIP * 8, 3 * H)[:ITEM_N])


# speedup vs baseline: 1.1060x; 1.1060x over previous
import functools

import jax
import jax.numpy as jnp
import numpy as np
from jax import lax
from jax.experimental import pallas as pl
from jax.experimental.pallas import tpu as pltpu
from jax.experimental.pallas import tpu_sc as plsc

USER_N = 60000
ITEM_N = 40000
NN = USER_N + ITEM_N
EDGES = 3200000
H = 16
SLOPE = 0.2

NC = 2
NS = 16
NW = NC * NS
CHUNK = 128
NCHUNKS = EDGES // CHUNK
BASE_CPW = NCHUNKS // NW
EXTRA = NCHUNKS - NW * BASE_CPW
SUP = 16
FULL_STEPS = BASE_CPW // SUP
TAIL = BASE_CPW - FULL_STEPS * SUP
NBUF = 8
NPAD = 102400
RPS = NPAD // NS
GROWS = NBUF * CHUNK

_BC_DNUMS = lax.GatherDimensionNumbers(
    offset_dims=(), collapsed_slice_dims=(0,), start_index_map=(0,))


def _bcast_lane(vec, lane):
    idx = jnp.full((16, 1), lane, jnp.int32)
    return lax.gather(vec, idx, _BC_DNUMS, (1,),
                      mode=lax.GatherScatterMode.PROMISE_IN_BOUNDS)


def _spmm_sc(rows2d, cols2d, vals2d, feat):
    mesh = plsc.VectorSubcoreMesh(core_axis_name="c", subcore_axis_name="s")

    @functools.partial(
        pl.kernel,
        out_type=jax.ShapeDtypeStruct((NC, NPAD, H), jnp.float32),
        mesh=mesh,
        scratch_types=[
            pltpu.VMEM((SUP, CHUNK), jnp.int32),
            pltpu.VMEM((SUP, CHUNK), jnp.int32),
            pltpu.VMEM((SUP, CHUNK), jnp.float32),
            pltpu.VMEM((GROWS, H), jnp.float32),
            pltpu.VMEM_SHARED((NPAD, H), jnp.float32),
            pltpu.SemaphoreType.DMA((NBUF,)),
            pltpu.SemaphoreType.DMA((NBUF,)),
        ],
        compiler_params=pltpu.CompilerParams(use_tc_tiling_on_sc=False),
    )
    def k(rows_hbm, cols_hbm, vals_hbm, feat_hbm, out_hbm,
          row_v, col_v, val_v, gat_v, acc, gsem, ssem):
        cid = lax.axis_index("c")
        sid = lax.axis_index("s")
        wid = cid * NS + sid

        @pl.loop(0, GROWS)
        def _(g):
            gat_v[g, :] = jnp.zeros((16,), jnp.float32)

        for t in range(RPS // GROWS):
            pltpu.sync_copy(gat_v, acc.at[pl.ds(sid * RPS + t * GROWS, GROWS)])
        rem = RPS % GROWS
        if rem:
            pltpu.sync_copy(
                gat_v.at[pl.ds(0, rem)],
                acc.at[pl.ds(sid * RPS + (RPS // GROWS) * GROWS, rem)])
        plsc.subcore_barrier()

        def process_staged(count):
            for r0 in range(0, count, NBUF):
                n = min(NBUF, count - r0)
                gets = [
                    pltpu.async_copy(feat_hbm.at[col_v.at[r0 + u]],
                                     gat_v.at[pl.ds(u * CHUNK, CHUNK)],
                                     gsem.at[u])
                    for u in range(n)
                ]
                puts = []
                for u in range(n):
                    j = r0 + u
                    gets[u].wait()

                    @plsc.parallel_loop(0, CHUNK // 16, unroll=2)
                    def _(g, u=u, j=j):
                        vv = val_v[j, pl.ds(g * 16, 16)]
                        for l in range(16):
                            bc = _bcast_lane(vv, l)
                            kk = u * CHUNK + g * 16 + l
                            gat_v[kk, :] = gat_v[kk, :] * bc

                    puts.append(
                        pltpu.async_copy(gat_v.at[pl.ds(u * CHUNK, CHUNK)],
                                         acc.at[row_v.at[j]],
                                         ssem.at[u], add=True))
                for p in puts:
                    p.wait()

        lo = wid * BASE_CPW + jnp.minimum(wid, EXTRA)

        @pl.loop(0, FULL_STEPS)
        def _(si):
            c0 = lo + si * SUP
            pltpu.sync_copy(rows_hbm.at[pl.ds(c0, SUP)], row_v)
            pltpu.sync_copy(cols_hbm.at[pl.ds(c0, SUP)], col_v)
            pltpu.sync_copy(vals_hbm.at[pl.ds(c0, SUP)], val_v)
            process_staged(SUP)

        ct = lo + FULL_STEPS * SUP
        pltpu.sync_copy(rows_hbm.at[pl.ds(ct, TAIL)], row_v.at[pl.ds(0, TAIL)])
        pltpu.sync_copy(cols_hbm.at[pl.ds(ct, TAIL)], col_v.at[pl.ds(0, TAIL)])
        pltpu.sync_copy(vals_hbm.at[pl.ds(ct, TAIL)], val_v.at[pl.ds(0, TAIL)])
        process_staged(TAIL)

        @pl.when(wid < EXTRA)
        def _():
            ce = lo + BASE_CPW
            pltpu.sync_copy(rows_hbm.at[pl.ds(ce, 1)], row_v.at[pl.ds(0, 1)])
            pltpu.sync_copy(cols_hbm.at[pl.ds(ce, 1)], col_v.at[pl.ds(0, 1)])
            pltpu.sync_copy(vals_hbm.at[pl.ds(ce, 1)], val_v.at[pl.ds(0, 1)])
            process_staged(1)

        plsc.subcore_barrier()
        pltpu.sync_copy(acc.at[pl.ds(sid * RPS, RPS)],
                        out_hbm.at[cid, pl.ds(sid * RPS, RPS)])

    return k(rows2d, cols2d, vals2d, feat)


PU = USER_N // 8
PI = ITEM_N // 8
PBLK = 512
PUP = 7680
PIP = 5120
UBLK = PUP // PBLK
NBLK = UBLK + PIP // PBLK
PT = NPAD // 8
ISHIFT = PUP * 8 - USER_N
_HP = lax.Precision.HIGHEST


def _seg16():
    r = lax.broadcasted_iota(jnp.int32, (128, 128), 0) // H
    c = lax.broadcasted_iota(jnp.int32, (128, 128), 1) // H
    return jnp.where(r == c, 1.0, 0.0).astype(jnp.float32)


def _kron8(w):
    eye = jnp.eye(8, dtype=jnp.float32)
    return jnp.einsum("ab,kc->akbc", eye, w).reshape(128, 128)


def _transform_tc(user_p, item_p, wu, wv):
    def body(u_ref, v_ref, wu_ref, wv_ref, out_ref):
        i = pl.program_id(0)
        x = jnp.where(i < UBLK, u_ref[...], v_ref[...])
        w = jnp.where(i < UBLK, wu_ref[...], wv_ref[...])
        out_ref[...] = jnp.dot(x, w, preferred_element_type=jnp.float32,
                               precision=_HP)

    return pl.pallas_call(
        body,
        grid=(NBLK,),
        in_specs=[
            pl.BlockSpec((PBLK, 128), lambda i: (jnp.minimum(i, UBLK - 1), 0)),
            pl.BlockSpec((PBLK, 128), lambda i: (jnp.maximum(i - UBLK, 0), 0)),
            pl.BlockSpec((128, 128), lambda i: (0, 0)),
            pl.BlockSpec((128, 128), lambda i: (0, 0)),
        ],
        out_specs=pl.BlockSpec((PBLK, 128), lambda i: (i, 0)),
        out_shape=jax.ShapeDtypeStruct((PT, 128), jnp.float32),
    )(user_p, item_p, wu, wv)


def _leaky_norm(s, seg):
    emb = jnp.where(s >= 0, s, SLOPE * s)
    n2 = jnp.dot(emb * emb, seg, preferred_element_type=jnp.float32,
                 precision=_HP)
    nrm = emb / jnp.maximum(jnp.sqrt(n2), 1e-12)
    return emb, nrm


def _combine_tc(p_p, wu, wv):
    def body(p_ref, wu_ref, wv_ref, nrm_ref, feat_ref):
        i = pl.program_id(0)
        emb, nrm = _leaky_norm(p_ref[0] + p_ref[1], _seg16())
        nrm_ref[...] = nrm
        w = jnp.where(i < UBLK, wu_ref[...], wv_ref[...])
        feat_ref[...] = jnp.dot(emb, w, preferred_element_type=jnp.float32,
                                precision=_HP)

    return pl.pallas_call(
        body,
        grid=(NBLK,),
        in_specs=[
            pl.BlockSpec((NC, PBLK, 128), lambda i: (0, i, 0)),
            pl.BlockSpec((128, 128), lambda i: (0, 0)),
            pl.BlockSpec((128, 128), lambda i: (0, 0)),
        ],
        out_specs=[pl.BlockSpec((PBLK, 128), lambda i: (i, 0)),
                   pl.BlockSpec((PBLK, 128), lambda i: (i, 0))],
        out_shape=[jax.ShapeDtypeStruct((PT, 128), jnp.float32),
                   jax.ShapeDtypeStruct((PT, 128), jnp.float32)],
    )(p_p, wu, wv)


def _final_tc(p_p, nrm1_p, emb_part_p, smat, blk0, nblk):
    def body(p_ref, n1_ref, e_ref, s_ref, out_ref):
        _, nrm2 = _leaky_norm(p_ref[0] + p_ref[1], _seg16())
        hp = dict(preferred_element_type=jnp.float32, precision=_HP)
        out_ref[...] = (jnp.dot(e_ref[...], s_ref[0], **hp) +
                        jnp.dot(n1_ref[...], s_ref[1], **hp) +
                        jnp.dot(nrm2, s_ref[2], **hp))

    return pl.pallas_call(
        body,
        grid=(nblk,),
        in_specs=[
            pl.BlockSpec((NC, PBLK, 128), lambda i: (0, i + blk0, 0)),
            pl.BlockSpec((PBLK, 128), lambda i: (i + blk0, 0)),
            pl.BlockSpec((PBLK, 128), lambda i: (i, 0)),
            pl.BlockSpec((3, 128, 384), lambda i: (0, 0, 0)),
        ],
        out_specs=pl.BlockSpec((PBLK, 384), lambda i: (i, 0)),
        out_shape=jax.ShapeDtypeStruct((nblk * PBLK, 384), jnp.float32),
    )(p_p, nrm1_p, emb_part_p, smat)


def _smat():
    m = np.zeros((3, 128, 384), np.float32)
    for t in range(3):
        for mm in range(8):
            for c in range(H):
                m[t, mm * H + c, mm * 3 * H + t * H + c] = 1.0
    return jnp.asarray(m)


def kernel(adj_indices, adj_values, user_emb, item_emb, u_w0, v_w0, u_w1, v_w1):
    idx = adj_indices + jnp.where(adj_indices >= USER_N, ISHIFT, 0)
    rows2d = idx[0].reshape(NCHUNKS, CHUNK)
    cols2d = idx[1].reshape(NCHUNKS, CHUNK)
    vals2d = adj_values.reshape(NCHUNKS, CHUNK)

    zu = jnp.zeros((PUP - PU, 128), jnp.float32)
    zi = jnp.zeros((PIP - PI, 128), jnp.float32)
    user_p = jnp.concatenate([user_emb.reshape(PU, 128), zu], axis=0)
    item_p = jnp.concatenate([item_emb.reshape(PI, 128), zi], axis=0)
    w0u, w0v, w1u, w1v = (_kron8(w) for w in (u_w0, v_w0, u_w1, v_w1))
    smat = _smat()

    table0 = _transform_tc(user_p, item_p, w0u, w0v)

    p1 = _spmm_sc(rows2d, cols2d, vals2d, table0.reshape(NPAD, H))
    nrm1, table1 = _combine_tc(p1.reshape(NC, PT, 128), w1u, w1v)

    p2 = _spmm_sc(rows2d, cols2d, vals2d, table1.reshape(NPAD, H))
    p2_p = p2.reshape(NC, PT, 128)

    user_embedding = _final_tc(p2_p, nrm1, user_p, smat, 0, UBLK)
    item_embedding = _final_tc(p2_p, nrm1, item_p, smat, UBLK, NBLK - UBLK)
    return (user_embedding.reshape(PUP * 8, 3 * H)[:USER_N],
            item_embedding.reshape(PIP * 8, 3 * H)[:ITEM_N])


# speedup vs baseline: 1.1478x; 1.0378x over previous
import functools

import jax
import jax.numpy as jnp
import numpy as np
from jax import lax
from jax.experimental import pallas as pl
from jax.experimental.pallas import tpu as pltpu
from jax.experimental.pallas import tpu_sc as plsc

USER_N = 60000
ITEM_N = 40000
NN = USER_N + ITEM_N
EDGES = 3200000
H = 16
SLOPE = 0.2

NC = 2
NS = 16
NW = NC * NS
CHUNK = 128
NCHUNKS = EDGES // CHUNK
BASE_CPW = NCHUNKS // NW
EXTRA = NCHUNKS - NW * BASE_CPW
SUP = 16
FULL_STEPS = BASE_CPW // SUP
TAIL = BASE_CPW - FULL_STEPS * SUP
NBUF = 8
NPAD = 102400
RPS = NPAD // NS
GROWS = NBUF * CHUNK

_BC_DNUMS = lax.GatherDimensionNumbers(
    offset_dims=(), collapsed_slice_dims=(0,), start_index_map=(0,))


def _bcast_lane(vec, lane):
    idx = jnp.full((16, 1), lane, jnp.int32)
    return lax.gather(vec, idx, _BC_DNUMS, (1,),
                      mode=lax.GatherScatterMode.PROMISE_IN_BOUNDS)


def _spmm_sc(rows2d, cols2d, vals2d, feat):
    mesh = plsc.VectorSubcoreMesh(core_axis_name="c", subcore_axis_name="s")

    @functools.partial(
        pl.kernel,
        out_type=jax.ShapeDtypeStruct((NC, NPAD, H), jnp.float32),
        mesh=mesh,
        scratch_types=[
            pltpu.VMEM((SUP, CHUNK), jnp.int32),
            pltpu.VMEM((SUP, CHUNK), jnp.int32),
            pltpu.VMEM((SUP, CHUNK), jnp.float32),
            pltpu.VMEM((GROWS, H), jnp.float32),
            pltpu.VMEM_SHARED((NPAD, H), jnp.float32),
            pltpu.SemaphoreType.DMA((NBUF,)),
            pltpu.SemaphoreType.DMA((NBUF,)),
        ],
        compiler_params=pltpu.CompilerParams(use_tc_tiling_on_sc=False),
    )
    def k(rows_hbm, cols_hbm, vals_hbm, feat_hbm, out_hbm,
          row_v, col_v, val_v, gat_v, acc, gsem, ssem):
        cid = lax.axis_index("c")
        sid = lax.axis_index("s")
        wid = cid * NS + sid

        @pl.loop(0, GROWS)
        def _(g):
            gat_v[g, :] = jnp.zeros((16,), jnp.float32)

        for t in range(RPS // GROWS):
            pltpu.sync_copy(gat_v, acc.at[pl.ds(sid * RPS + t * GROWS, GROWS)])
        rem = RPS % GROWS
        if rem:
            pltpu.sync_copy(
                gat_v.at[pl.ds(0, rem)],
                acc.at[pl.ds(sid * RPS + (RPS // GROWS) * GROWS, rem)])
        plsc.subcore_barrier()

        HB = NBUF // 2

        def fire_gathers(grp, h):
            return [
                pltpu.async_copy(feat_hbm.at[col_v.at[j]],
                                 gat_v.at[pl.ds((h * HB + k) * CHUNK, CHUNK)],
                                 gsem.at[h * HB + k])
                for k, j in enumerate(grp)
            ]

        def process_staged(count):
            groups = [list(range(s, min(s + HB, count)))
                      for s in range(0, count, HB)]
            pend_s = [None, None]
            pend_g = fire_gathers(groups[0], 0)
            for gi, grp in enumerate(groups):
                h = gi % 2
                oh = 1 - h
                if gi + 1 < len(groups):
                    if pend_s[oh] is not None:
                        for p in pend_s[oh]:
                            p.wait()
                    ng = fire_gathers(groups[gi + 1], oh)
                puts = []
                for k, j in enumerate(grp):
                    u = h * HB + k
                    pend_g[k].wait()

                    @pl.loop(0, CHUNK // 16)
                    def _(g, u=u, j=j):
                        vv = val_v[j, pl.ds(g * 16, 16)]
                        for l in range(16):
                            bc = _bcast_lane(vv, l)
                            kk = u * CHUNK + g * 16 + l
                            gat_v[kk, :] = gat_v[kk, :] * bc

                    puts.append(
                        pltpu.async_copy(gat_v.at[pl.ds(u * CHUNK, CHUNK)],
                                         acc.at[row_v.at[j]],
                                         ssem.at[u], add=True))
                pend_s[h] = puts
                if gi + 1 < len(groups):
                    pend_g = ng
            for ps in pend_s:
                if ps is not None:
                    for p in ps:
                        p.wait()

        lo = wid * BASE_CPW + jnp.minimum(wid, EXTRA)

        @pl.loop(0, FULL_STEPS)
        def _(si):
            c0 = lo + si * SUP
            pltpu.sync_copy(rows_hbm.at[pl.ds(c0, SUP)], row_v)
            pltpu.sync_copy(cols_hbm.at[pl.ds(c0, SUP)], col_v)
            pltpu.sync_copy(vals_hbm.at[pl.ds(c0, SUP)], val_v)
            process_staged(SUP)

        ct = lo + FULL_STEPS * SUP
        pltpu.sync_copy(rows_hbm.at[pl.ds(ct, TAIL)], row_v.at[pl.ds(0, TAIL)])
        pltpu.sync_copy(cols_hbm.at[pl.ds(ct, TAIL)], col_v.at[pl.ds(0, TAIL)])
        pltpu.sync_copy(vals_hbm.at[pl.ds(ct, TAIL)], val_v.at[pl.ds(0, TAIL)])
        process_staged(TAIL)

        @pl.when(wid < EXTRA)
        def _():
            ce = lo + BASE_CPW
            pltpu.sync_copy(rows_hbm.at[pl.ds(ce, 1)], row_v.at[pl.ds(0, 1)])
            pltpu.sync_copy(cols_hbm.at[pl.ds(ce, 1)], col_v.at[pl.ds(0, 1)])
            pltpu.sync_copy(vals_hbm.at[pl.ds(ce, 1)], val_v.at[pl.ds(0, 1)])
            process_staged(1)

        plsc.subcore_barrier()
        pltpu.sync_copy(acc.at[pl.ds(sid * RPS, RPS)],
                        out_hbm.at[cid, pl.ds(sid * RPS, RPS)])

    return k(rows2d, cols2d, vals2d, feat)


PU = USER_N // 8
PI = ITEM_N // 8
PBLK = 512
PUP = 7680
PIP = 5120
UBLK = PUP // PBLK
NBLK = UBLK + PIP // PBLK
PT = NPAD // 8
ISHIFT = PUP * 8 - USER_N
_HP = lax.Precision.HIGHEST


def _seg16():
    r = lax.broadcasted_iota(jnp.int32, (128, 128), 0) // H
    c = lax.broadcasted_iota(jnp.int32, (128, 128), 1) // H
    return jnp.where(r == c, 1.0, 0.0).astype(jnp.float32)


def _kron8(w):
    eye = jnp.eye(8, dtype=jnp.float32)
    return jnp.einsum("ab,kc->akbc", eye, w).reshape(128, 128)


def _transform_tc(user_p, item_p, wu, wv):
    def body(u_ref, v_ref, wu_ref, wv_ref, out_ref):
        i = pl.program_id(0)
        x = jnp.where(i < UBLK, u_ref[...], v_ref[...])
        w = jnp.where(i < UBLK, wu_ref[...], wv_ref[...])
        out_ref[...] = jnp.dot(x, w, preferred_element_type=jnp.float32,
                               precision=_HP)

    return pl.pallas_call(
        body,
        grid=(NBLK,),
        in_specs=[
            pl.BlockSpec((PBLK, 128), lambda i: (jnp.minimum(i, UBLK - 1), 0)),
            pl.BlockSpec((PBLK, 128), lambda i: (jnp.maximum(i - UBLK, 0), 0)),
            pl.BlockSpec((128, 128), lambda i: (0, 0)),
            pl.BlockSpec((128, 128), lambda i: (0, 0)),
        ],
        out_specs=pl.BlockSpec((PBLK, 128), lambda i: (i, 0)),
        out_shape=jax.ShapeDtypeStruct((PT, 128), jnp.float32),
    )(user_p, item_p, wu, wv)


def _leaky_norm(s, seg):
    emb = jnp.where(s >= 0, s, SLOPE * s)
    n2 = jnp.dot(emb * emb, seg, preferred_element_type=jnp.float32,
                 precision=_HP)
    nrm = emb / jnp.maximum(jnp.sqrt(n2), 1e-12)
    return emb, nrm


def _combine_tc(p_p, wu, wv):
    def body(p_ref, wu_ref, wv_ref, nrm_ref, feat_ref):
        i = pl.program_id(0)
        emb, nrm = _leaky_norm(p_ref[0] + p_ref[1], _seg16())
        nrm_ref[...] = nrm
        w = jnp.where(i < UBLK, wu_ref[...], wv_ref[...])
        feat_ref[...] = jnp.dot(emb, w, preferred_element_type=jnp.float32,
                                precision=_HP)

    return pl.pallas_call(
        body,
        grid=(NBLK,),
        in_specs=[
            pl.BlockSpec((NC, PBLK, 128), lambda i: (0, i, 0)),
            pl.BlockSpec((128, 128), lambda i: (0, 0)),
            pl.BlockSpec((128, 128), lambda i: (0, 0)),
        ],
        out_specs=[pl.BlockSpec((PBLK, 128), lambda i: (i, 0)),
                   pl.BlockSpec((PBLK, 128), lambda i: (i, 0))],
        out_shape=[jax.ShapeDtypeStruct((PT, 128), jnp.float32),
                   jax.ShapeDtypeStruct((PT, 128), jnp.float32)],
    )(p_p, wu, wv)


def _final_tc(p_p, nrm1_p, emb_part_p, smat, blk0, nblk):
    def body(p_ref, n1_ref, e_ref, s_ref, out_ref):
        _, nrm2 = _leaky_norm(p_ref[0] + p_ref[1], _seg16())
        hp = dict(preferred_element_type=jnp.float32, precision=_HP)
        out_ref[...] = (jnp.dot(e_ref[...], s_ref[0], **hp) +
                        jnp.dot(n1_ref[...], s_ref[1], **hp) +
                        jnp.dot(nrm2, s_ref[2], **hp))

    return pl.pallas_call(
        body,
        grid=(nblk,),
        in_specs=[
            pl.BlockSpec((NC, PBLK, 128), lambda i: (0, i + blk0, 0)),
            pl.BlockSpec((PBLK, 128), lambda i: (i + blk0, 0)),
            pl.BlockSpec((PBLK, 128), lambda i: (i, 0)),
            pl.BlockSpec((3, 128, 384), lambda i: (0, 0, 0)),
        ],
        out_specs=pl.BlockSpec((PBLK, 384), lambda i: (i, 0)),
        out_shape=jax.ShapeDtypeStruct((nblk * PBLK, 384), jnp.float32),
    )(p_p, nrm1_p, emb_part_p, smat)


def _smat():
    m = np.zeros((3, 128, 384), np.float32)
    for t in range(3):
        for mm in range(8):
            for c in range(H):
                m[t, mm * H + c, mm * 3 * H + t * H + c] = 1.0
    return jnp.asarray(m)


def kernel(adj_indices, adj_values, user_emb, item_emb, u_w0, v_w0, u_w1, v_w1):
    idx = adj_indices + jnp.where(adj_indices >= USER_N, ISHIFT, 0)
    rows2d = idx[0].reshape(NCHUNKS, CHUNK)
    cols2d = idx[1].reshape(NCHUNKS, CHUNK)
    vals2d = adj_values.reshape(NCHUNKS, CHUNK)

    zu = jnp.zeros((PUP - PU, 128), jnp.float32)
    zi = jnp.zeros((PIP - PI, 128), jnp.float32)
    user_p = jnp.concatenate([user_emb.reshape(PU, 128), zu], axis=0)
    item_p = jnp.concatenate([item_emb.reshape(PI, 128), zi], axis=0)
    w0u, w0v, w1u, w1v = (_kron8(w) for w in (u_w0, v_w0, u_w1, v_w1))
    smat = _smat()

    table0 = _transform_tc(user_p, item_p, w0u, w0v)

    p1 = _spmm_sc(rows2d, cols2d, vals2d, table0.reshape(NPAD, H))
    nrm1, table1 = _combine_tc(p1.reshape(NC, PT, 128), w1u, w1v)

    p2 = _spmm_sc(rows2d, cols2d, vals2d, table1.reshape(NPAD, H))
    p2_p = p2.reshape(NC, PT, 128)

    user_embedding = _final_tc(p2_p, nrm1, user_p, smat, 0, UBLK)
    item_embedding = _final_tc(p2_p, nrm1, item_p, smat, UBLK, NBLK - UBLK)
    return (user_embedding.reshape(PUP * 8, 3 * H)[:USER_N],
            item_embedding.reshape(PIP * 8, 3 * H)[:ITEM_N])
